# baseline (speedup 1.0000x reference)
import jax
import jax.numpy as jnp
from jax.experimental import pallas as pl

B_TILE = 512


def _onehot_block(t_ref, out_ref):
    tb = t_ref[...]
    cls = jax.lax.broadcasted_iota(jnp.int32, out_ref.shape, 1)
    out_ref[...] = (tb == cls).astype(jnp.float32)


def kernel(t, ones):
    B, S = t.shape
    C = ones.shape[0]
    t3 = t.astype(jnp.int32).T.reshape(S, 1, B)
    out_t = pl.pallas_call(
        _onehot_block,
        grid=(S, B // B_TILE),
        in_specs=[pl.BlockSpec((1, 1, B_TILE), lambda s, j: (s, 0, j))],
        out_specs=pl.BlockSpec((1, C, B_TILE), lambda s, j: (s, 0, j)),
        out_shape=jax.ShapeDtypeStruct((S, C, B), jnp.float32),
    )(t3)
    return jnp.transpose(out_t, (2, 1, 0))


# speedup vs baseline: 1.3090x; 1.3090x over previous
import jax
import jax.numpy as jnp
from jax.experimental import pallas as pl

B_TILE = 4096


def _onehot_block(t_ref, out_ref):
    tb = t_ref[...]
    cls = jax.lax.broadcasted_iota(jnp.int32, out_ref.shape, 1)
    out_ref[...] = (tb == cls).astype(jnp.float32)


def kernel(t, ones):
    B, S = t.shape
    C = ones.shape[0]
    t3 = t.astype(jnp.int32).T.reshape(S, 1, B)
    out_t = pl.pallas_call(
        _onehot_block,
        grid=(S, B // B_TILE),
        in_specs=[pl.BlockSpec((1, 1, B_TILE), lambda s, j: (s, 0, j))],
        out_specs=pl.BlockSpec((1, C, B_TILE), lambda s, j: (s, 0, j)),
        out_shape=jax.ShapeDtypeStruct((S, C, B), jnp.float32),
    )(t3)
    return jnp.transpose(out_t, (2, 1, 0))


# speedup vs baseline: 1.3468x; 1.0288x over previous
import jax
import jax.numpy as jnp
from jax.experimental import pallas as pl

B_TILE = 1024


def _onehot_block(t_ref, out_ref):
    tb = t_ref[...]
    cls = jax.lax.broadcasted_iota(jnp.int32, out_ref.shape, 1)
    out_ref[...] = (tb == cls).astype(jnp.float32)


def kernel(t, ones):
    B, S = t.shape
    C = ones.shape[0]
    t3 = t.astype(jnp.int32).T.reshape(S, 1, B)
    out_t = pl.pallas_call(
        _onehot_block,
        grid=(S, B // B_TILE),
        in_specs=[pl.BlockSpec((1, 1, B_TILE), lambda s, j: (s, 0, j))],
        out_specs=pl.BlockSpec((1, C, B_TILE), lambda s, j: (s, 0, j)),
        out_shape=jax.ShapeDtypeStruct((S, C, B), jnp.float32),
    )(t3)
    return jnp.transpose(out_t, (2, 1, 0))
